# Initial kernel scaffold; baseline (speedup 1.0000x reference)
import jax
import jax.numpy as jnp
from jax.experimental import pallas as pl


def kernel(h, edge_index, edge_weight, node_types, t_emb, W0, b0, W1, b1, W2, b2):
    raise NotImplementedError("write your pallas kernel here")


# speedup vs baseline: 5.7330x; 5.7330x over previous
import functools

import jax
import jax.numpy as jnp
from jax import lax
from jax.experimental import pallas as pl
from jax.experimental.pallas import tpu as pltpu
from jax.experimental.pallas import tpu_sc as plsc

NC = 2
NS = 16
C = 128

HD = 128
TE = 64


def _sc_scatter(h_hbm, src_hbm, dst_hbm, w_hbm, zrow_hbm, zdeg_hbm,
                agg_out, deg_out,
                s_idx, d_idx, w_v, rows, agg_sh, deg_sh, gsem):
    n_pad = agg_sh.shape[0]
    nchunks = src_hbm.shape[0]
    per_core = nchunks // NC
    rows_per_tile = n_pad // NS

    c = lax.axis_index("c")
    s = lax.axis_index("s")

    r0 = s * rows_per_tile
    for t in range(rows_per_tile // C):
        pltpu.sync_copy(zrow_hbm, agg_sh.at[pl.ds(r0 + t * C, C)])
    pltpu.sync_copy(zdeg_hbm, deg_sh.at[pl.ds(r0, rows_per_tile)])
    plsc.subcore_barrier()

    max_iters = (per_core + NS - 1) // NS

    def chunk_body(k, _):
        rel = s + k * NS

        @pl.when(rel < per_core)
        def _():
            ch = c * per_core + rel
            pltpu.sync_copy(src_hbm.at[ch], s_idx.at[0])
            pltpu.sync_copy(dst_hbm.at[ch], d_idx.at[0])
            pltpu.sync_copy(w_hbm.at[ch], w_v.at[0])
            pltpu.async_copy(h_hbm.at[s_idx.at[0]], rows, gsem).wait()

            def scale_group(t, carry):
                wv = w_v[0, pl.ds(t * 16, 16)]
                for e16 in range(16):
                    w = wv[e16]
                    e = t * 16 + e16
                    for q in range(HD // 16):
                        sl = pl.ds(q * 16, 16)
                        rows[e, sl] = rows[e, sl] * w
                return carry

            lax.fori_loop(0, C // 16, scale_group, 0)
            pltpu.sync_copy(rows, agg_sh.at[d_idx.at[0]], add=True)
            pltpu.sync_copy(w_v.at[0], deg_sh.at[d_idx.at[0]], add=True)

        return k

    lax.fori_loop(0, max_iters, chunk_body, 0)
    plsc.subcore_barrier()

    for t in range(rows_per_tile // C):
        sl = pl.ds(r0 + t * C, C)
        pltpu.sync_copy(agg_sh.at[sl], agg_out.at[c].at[sl])
    pltpu.sync_copy(deg_sh.at[pl.ds(r0, rows_per_tile)],
                    deg_out.at[c].at[pl.ds(r0, rows_per_tile)])


def _sc_aggregate(h, src2d, dst2d, w2d, n_pad):
    nchunks = src2d.shape[0]
    mesh = plsc.VectorSubcoreMesh(core_axis_name="c", subcore_axis_name="s")
    zrow = jnp.zeros((C, HD), jnp.float32)
    zdeg = jnp.zeros((n_pad // NS,), jnp.float32)
    f = pl.kernel(
        _sc_scatter,
        out_type=[
            jax.ShapeDtypeStruct((NC, n_pad, HD), jnp.float32),
            jax.ShapeDtypeStruct((NC, n_pad), jnp.float32),
        ],
        mesh=mesh,
        scratch_types=[
            pltpu.VMEM((1, C), jnp.int32),
            pltpu.VMEM((1, C), jnp.int32),
            pltpu.VMEM((1, C), jnp.float32),
            pltpu.VMEM((C, HD), jnp.float32),
            pltpu.VMEM_SHARED((n_pad, HD), jnp.float32),
            pltpu.VMEM_SHARED((n_pad,), jnp.float32),
            pltpu.SemaphoreType.DMA,
        ],
    )
    return f(h, src2d, dst2d, w2d, zrow, zdeg)


def _tc_mlp_body(h_ref, p_ref, d_ref, nt_ref, t_ref, wh_ref, wa_ref, wt_ref,
                 b_ref, o_ref):
    p = p_ref[0] + p_ref[1]
    d = d_ref[0] + d_ref[1]
    agg = p / jnp.maximum(d, 1e-8)
    x = (jnp.dot(h_ref[...], wh_ref[...], preferred_element_type=jnp.float32)
         + jnp.dot(agg, wa_ref[...], preferred_element_type=jnp.float32)
         + jnp.dot(t_ref[...], wt_ref[...], preferred_element_type=jnp.float32)
         + b_ref[...])
    g = 0.5 * x * (1.0 + lax.erf(x * 0.7071067811865476))
    nt = nt_ref[...]
    out = jnp.where(nt == 0, g[:, :HD],
                    jnp.where(nt == 1, g[:, HD:2 * HD], g[:, 2 * HD:]))
    o_ref[...] = out


def _tc_mlp(h, agg_parts, deg_parts, node_types, t_emb, wh, wa, wt, b):
    n = h.shape[0]
    r = 2048
    grid = (n + r - 1) // r
    return pl.pallas_call(
        _tc_mlp_body,
        grid=(grid,),
        in_specs=[
            pl.BlockSpec((r, HD), lambda i: (i, 0)),
            pl.BlockSpec((NC, r, HD), lambda i: (0, i, 0)),
            pl.BlockSpec((NC, r, 1), lambda i: (0, i, 0)),
            pl.BlockSpec((r, 1), lambda i: (i, 0)),
            pl.BlockSpec((1, TE), lambda i: (0, 0)),
            pl.BlockSpec((HD, 3 * HD), lambda i: (0, 0)),
            pl.BlockSpec((HD, 3 * HD), lambda i: (0, 0)),
            pl.BlockSpec((TE, 3 * HD), lambda i: (0, 0)),
            pl.BlockSpec((1, 3 * HD), lambda i: (0, 0)),
        ],
        out_specs=pl.BlockSpec((r, HD), lambda i: (i, 0)),
        out_shape=jax.ShapeDtypeStruct((n, HD), jnp.float32),
    )(h, agg_parts, deg_parts, node_types, t_emb, wh, wa, wt, b)


def kernel(h, edge_index, edge_weight, node_types, t_emb,
           W0, b0, W1, b1, W2, b2):
    n = h.shape[0]
    e = edge_weight.shape[0]
    n_pad = ((n + NS * C - 1) // (NS * C)) * (NS * C)
    nchunks = e // C

    src2d = edge_index[0].reshape(nchunks, C)
    dst2d = edge_index[1].reshape(nchunks, C)
    w2d = edge_weight.reshape(nchunks, C)

    agg_parts, deg_parts = _sc_aggregate(h, src2d, dst2d, w2d, n_pad)

    wc = jnp.concatenate([W0, W1, W2], axis=1)
    wh = wc[:HD]
    wa = wc[HD:2 * HD]
    wt = wc[2 * HD:]
    b = jnp.concatenate([b0, b1, b2]).reshape(1, 3 * HD)

    return _tc_mlp(
        h,
        agg_parts,
        deg_parts.reshape(NC, n_pad, 1),
        node_types.reshape(n, 1),
        t_emb.reshape(1, TE),
        wh, wa, wt, b,
    )
